# Initial kernel scaffold; baseline (speedup 1.0000x reference)
import jax
import jax.numpy as jnp
from jax.experimental import pallas as pl


def kernel(h, e, edge_index, w, x, params):
    raise NotImplementedError("write your pallas kernel here")


# speedup vs baseline: 2.1126x; 2.1126x over previous
import functools

import jax
import jax.numpy as jnp
from jax import lax
from jax.experimental import pallas as pl
from jax.experimental.pallas import tpu as pltpu
from jax.experimental.pallas import tpu_sc as plsc

N = 10000
E = 320000
D = 128
LAM = 1.0

NC = 2
NS = 16
NW = NC * NS
EPW = E // NW
CH = 80
NCH = EPW // CH

_MESH = plsc.VectorSubcoreMesh(core_axis_name="c", subcore_axis_name="s")


def _gather_body(tbl_hbm, src_hbm, dst_hbm, outs_hbm, outd_hbm,
                 idx_s, idx_d, buf_s, buf_d, sem_s, sem_d):
    cid = lax.axis_index("c")
    sid = lax.axis_index("s")
    base = (cid * NS + sid) * EPW

    def body(ci, carry):
        off = base + ci * CH
        pltpu.sync_copy(src_hbm.at[pl.ds(off, CH)], idx_s)
        pltpu.sync_copy(dst_hbm.at[pl.ds(off, CH)], idx_d)
        cp1 = pltpu.async_copy(tbl_hbm.at[idx_s], buf_s, sem_s)
        cp2 = pltpu.async_copy(tbl_hbm.at[idx_d], buf_d, sem_d)
        cp1.wait()
        cp2.wait()
        pltpu.sync_copy(buf_s, outs_hbm.at[pl.ds(off, CH)])
        pltpu.sync_copy(buf_d, outd_hbm.at[pl.ds(off, CH)])
        return carry

    lax.fori_loop(0, NCH, body, 0)


_gather = pl.kernel(
    _gather_body,
    out_type=[jax.ShapeDtypeStruct((E, D), jnp.float32),
              jax.ShapeDtypeStruct((E, D), jnp.float32)],
    mesh=_MESH,
    scratch_types=[
        pltpu.VMEM((CH,), jnp.int32),
        pltpu.VMEM((CH,), jnp.int32),
        pltpu.VMEM((CH, D), jnp.float32),
        pltpu.VMEM((CH, D), jnp.float32),
        pltpu.SemaphoreType.DMA,
        pltpu.SemaphoreType.DMA,
    ],
)


def _scatter_body(dual_hbm, dst_hbm, zeros_hbm, out_hbm, idx, buf, acc):
    cid = lax.axis_index("c")
    sid = lax.axis_index("s")

    @pl.when(sid == 0)
    def _():
        pltpu.sync_copy(zeros_hbm, acc)

    plsc.subcore_barrier()
    base = (cid * NS + sid) * EPW

    def body(ci, carry):
        off = base + ci * CH
        pltpu.sync_copy(dst_hbm.at[pl.ds(off, CH)], idx)
        pltpu.sync_copy(dual_hbm.at[pl.ds(off, CH)], buf)
        pltpu.sync_copy(buf, acc.at[idx], add=True)
        return carry

    lax.fori_loop(0, NCH, body, 0)
    plsc.subcore_barrier()

    @pl.when(sid < 10)
    def _():
        r0 = sid * 1000
        pltpu.sync_copy(acc.at[pl.ds(r0, 1000)],
                        out_hbm.at[cid].at[pl.ds(r0, 1000)])


_scatter = pl.kernel(
    _scatter_body,
    out_type=jax.ShapeDtypeStruct((NC, N, D), jnp.float32),
    mesh=_MESH,
    scratch_types=[
        pltpu.VMEM((CH,), jnp.int32),
        pltpu.VMEM((CH, D), jnp.float32),
        pltpu.VMEM_SHARED((N, D), jnp.float32),
    ],
)


def _dot(a, b):
    return lax.dot_general(a, b, (((1,), (0,)), ((), ())),
                           preferred_element_type=jnp.float32,
                           precision=lax.Precision.HIGHEST)


def _hw_body(h_ref, w_ref, o_ref):
    o_ref[...] = _dot(h_ref[...], w_ref[...])


TN = 1000


def _hw_mm(h, W):
    return pl.pallas_call(
        _hw_body,
        grid=(N // TN,),
        in_specs=[pl.BlockSpec((TN, D), lambda i: (i, 0)),
                  pl.BlockSpec((D, D), lambda i: (0, 0))],
        out_specs=pl.BlockSpec((TN, D), lambda i: (i, 0)),
        out_shape=jax.ShapeDtypeStruct((N, D), jnp.float32),
    )(h, W)


TE = 2000


def _edge_body(e_ref, gs_ref, gd_ref, w_ref, weu_ref, b_ref,
               eproj_ref, dual_ref):
    sqrtw = jnp.sqrt(w_ref[...])
    u = _dot(e_ref[...], weu_ref[...])
    u = u + b_ref[...] + sqrtw * (gs_ref[...] - gd_ref[...])
    norm = jnp.sqrt(jnp.sum(u * u, axis=1, keepdims=True))
    scale = jnp.minimum(1.0, (LAM * sqrtw) / (norm + 1e-12))
    ep = u * scale
    eproj_ref[...] = ep
    dual_ref[...] = sqrtw * ep


def _edge_stage(e, gs, gd, w2, W_eu, b_edge):
    return pl.pallas_call(
        _edge_body,
        grid=(E // TE,),
        in_specs=[pl.BlockSpec((TE, D), lambda i: (i, 0)),
                  pl.BlockSpec((TE, D), lambda i: (i, 0)),
                  pl.BlockSpec((TE, D), lambda i: (i, 0)),
                  pl.BlockSpec((TE, 1), lambda i: (i, 0)),
                  pl.BlockSpec((D, D), lambda i: (0, 0)),
                  pl.BlockSpec((1, D), lambda i: (0, 0))],
        out_specs=[pl.BlockSpec((TE, D), lambda i: (i, 0)),
                   pl.BlockSpec((TE, D), lambda i: (i, 0))],
        out_shape=[jax.ShapeDtypeStruct((E, D), jnp.float32),
                   jax.ShapeDtypeStruct((E, D), jnp.float32)],
    )(e, gs, gd, w2, W_eu, b_edge)


def _node_body(h_ref, x_ref, aggp_ref, wnf, wres, wagg, wf1, wf2,
               bn, bf1, bf2, o_ref):
    agg = aggp_ref[0] + aggp_ref[1]
    ni = (_dot(h_ref[...], wnf[...]) + _dot(x_ref[...], wres[...])
          + _dot(agg, wagg[...]) + bn[...])
    hid = jax.nn.silu(_dot(ni, wf1[...]) + bf1[...])
    o_ref[...] = _dot(hid, wf2[...]) + bf2[...]


def _node_stage(h, x, aggp, p):
    wspec = pl.BlockSpec((D, D), lambda i: (0, 0))
    bspec = pl.BlockSpec((1, D), lambda i: (0, 0))
    bn = (p['b_nf'] + p['b_res'] + p['b_agg']).reshape(1, D)
    return pl.pallas_call(
        _node_body,
        grid=(N // TN,),
        in_specs=[pl.BlockSpec((TN, D), lambda i: (i, 0)),
                  pl.BlockSpec((TN, D), lambda i: (i, 0)),
                  pl.BlockSpec((NC, TN, D), lambda i: (0, i, 0)),
                  wspec, wspec, wspec, wspec, wspec,
                  bspec, bspec, bspec],
        out_specs=pl.BlockSpec((TN, D), lambda i: (i, 0)),
        out_shape=jax.ShapeDtypeStruct((N, D), jnp.float32),
    )(h, x, aggp, p['W_nf'], p['W_res'], p['W_agg'], p['W_f1'], p['W_f2'],
      bn, p['b_f1'].reshape(1, D), p['b_f2'].reshape(1, D))


def kernel(h, e, edge_index, w, x, params):
    src = edge_index[0]
    dst = edge_index[1]
    w2 = w.reshape(E, 1)
    zeros = jnp.zeros((N, D), jnp.float32)
    for p in params:
        hW = _hw_mm(h, p['W_ea'])
        gs, gd = _gather(hW, src, dst)
        b_edge = (p['b_eu'] + p['b_ea']).reshape(1, D)
        e, dual = _edge_stage(e, gs, gd, w2, p['W_eu'], b_edge)
        aggp = _scatter(dual, dst, zeros)
        h = _node_stage(h, x, aggp, p)
    return (h, e)


# speedup vs baseline: 4.1588x; 1.9686x over previous
import jax
import jax.numpy as jnp
from jax import lax
from jax.experimental import pallas as pl
from jax.experimental.pallas import tpu as pltpu
from jax.experimental.pallas import tpu_sc as plsc

N = 10000
E = 320000
D = 128
LAM = 1.0

NC = 2
NS = 16
NW = NC * NS
EPW = E // NW
CH = 80
NCH = EPW // CH
NPAIR = (NCH - 1) // 2

_MESH = plsc.VectorSubcoreMesh(core_axis_name="c", subcore_axis_name="s")


def _gather_body(tbl, src3, dst3, g_hbm, idxs, idxd, bufs, bufd, bufg,
                 sem_s0, sem_s1, sem_d0, sem_d1, sem_w):
    cid = lax.axis_index("c")
    sid = lax.axis_index("s")
    wid = cid * NS + sid
    pltpu.sync_copy(src3.at[wid], idxs)
    pltpu.sync_copy(dst3.at[wid], idxd)
    sems = (sem_s0, sem_s1)
    semd = (sem_d0, sem_d1)

    def issue(c, k):
        pltpu.async_copy(tbl.at[idxs.at[c]], bufs.at[k], sems[k])
        pltpu.async_copy(tbl.at[idxd.at[c]], bufd.at[k], semd[k])

    def wait_gather(k):
        pltpu.make_async_copy(tbl.at[idxs.at[0]], bufs.at[k], sems[k]).wait()
        pltpu.make_async_copy(tbl.at[idxd.at[0]], bufd.at[k], semd[k]).wait()

    def sub(k):
        bs, bd, bg = bufs.at[k], bufd.at[k], bufg.at[k]

        def row(r, carry):
            for j in range(D // 16):
                sl = pl.ds(j * 16, 16)
                bg[r, sl] = bs[r, sl] - bd[r, sl]
            return carry

        lax.fori_loop(0, CH, row, 0)

    def write(c, k):
        off = (wid * NCH + c) * CH
        pltpu.async_copy(bufg.at[k], g_hbm.at[pl.ds(off, CH)], sem_w)

    def wait_write():
        pltpu.make_async_copy(bufg.at[0], g_hbm.at[pl.ds(0, CH)],
                              sem_w).wait()

    issue(0, 0)
    issue(1, 1)

    def body(i, carry):
        c0 = 2 * i
        wait_gather(0)

        @pl.when(i > 0)
        def _():
            wait_write()
            wait_write()

        sub(0)
        issue(c0 + 2, 0)
        write(c0, 0)
        wait_gather(1)
        sub(1)

        @pl.when(c0 + 3 < NCH)
        def _():
            issue(c0 + 3, 1)

        write(c0 + 1, 1)
        return carry

    lax.fori_loop(0, NPAIR, body, 0)
    wait_gather(0)
    wait_write()
    wait_write()
    sub(0)
    write(NCH - 1, 0)
    wait_write()


_gather = pl.kernel(
    _gather_body,
    out_type=jax.ShapeDtypeStruct((E, D), jnp.float32),
    mesh=_MESH,
    scratch_types=[
        pltpu.VMEM((NCH, CH), jnp.int32),
        pltpu.VMEM((NCH, CH), jnp.int32),
        pltpu.VMEM((2, CH, D), jnp.float32),
        pltpu.VMEM((2, CH, D), jnp.float32),
        pltpu.VMEM((2, CH, D), jnp.float32),
        pltpu.SemaphoreType.DMA,
        pltpu.SemaphoreType.DMA,
        pltpu.SemaphoreType.DMA,
        pltpu.SemaphoreType.DMA,
        pltpu.SemaphoreType.DMA,
    ],
)


def _scatter_body(dual4, dst3, zeros_hbm, out_hbm, idxd, buf, acc,
                  sem_r0, sem_r1):
    cid = lax.axis_index("c")
    sid = lax.axis_index("s")
    wid = cid * NS + sid

    @pl.when(sid == 0)
    def _():
        pltpu.sync_copy(zeros_hbm, acc)

    pltpu.sync_copy(dst3.at[wid], idxd)
    plsc.subcore_barrier()
    semr = (sem_r0, sem_r1)

    def issue(c, k):
        pltpu.async_copy(dual4.at[wid, c], buf.at[k], semr[k])

    def wait_read(k):
        pltpu.make_async_copy(dual4.at[wid, 0], buf.at[k], semr[k]).wait()

    def scat(c, k):
        pltpu.sync_copy(buf.at[k], acc.at[idxd.at[c]], add=True)

    issue(0, 0)
    issue(1, 1)

    def body(i, carry):
        c0 = 2 * i
        wait_read(0)
        scat(c0, 0)
        issue(c0 + 2, 0)
        wait_read(1)
        scat(c0 + 1, 1)

        @pl.when(c0 + 3 < NCH)
        def _():
            issue(c0 + 3, 1)

        return carry

    lax.fori_loop(0, NPAIR, body, 0)
    wait_read(0)
    scat(NCH - 1, 0)
    plsc.subcore_barrier()

    @pl.when(sid < 10)
    def _():
        r0 = sid * 1000
        pltpu.sync_copy(acc.at[pl.ds(r0, 1000)],
                        out_hbm.at[cid].at[pl.ds(r0, 1000)])


_scatter = pl.kernel(
    _scatter_body,
    out_type=jax.ShapeDtypeStruct((NC, N, D), jnp.float32),
    mesh=_MESH,
    scratch_types=[
        pltpu.VMEM((NCH, CH), jnp.int32),
        pltpu.VMEM((2, CH, D), jnp.float32),
        pltpu.VMEM_SHARED((N, D), jnp.float32),
        pltpu.SemaphoreType.DMA,
        pltpu.SemaphoreType.DMA,
    ],
)


def _dot(a, b):
    return lax.dot_general(a, b, (((1,), (0,)), ((), ())),
                           preferred_element_type=jnp.float32)


def _hw_body(h_ref, w_ref, o_ref):
    o_ref[...] = _dot(h_ref[...], w_ref[...])


TN = 1000


def _hw_mm(h, W):
    return pl.pallas_call(
        _hw_body,
        grid=(N // TN,),
        in_specs=[pl.BlockSpec((TN, D), lambda i: (i, 0)),
                  pl.BlockSpec((D, D), lambda i: (0, 0))],
        out_specs=pl.BlockSpec((TN, D), lambda i: (i, 0)),
        out_shape=jax.ShapeDtypeStruct((N, D), jnp.float32),
    )(h, W)


TE = 2000


def _edge_body(e_ref, g_ref, w_ref, weu_ref, b_ref, eproj_ref, dual_ref):
    sqrtw = jnp.sqrt(w_ref[...])
    u = _dot(e_ref[...], weu_ref[...])
    u = u + b_ref[...] + sqrtw * g_ref[...]
    norm = jnp.sqrt(jnp.sum(u * u, axis=1, keepdims=True))
    scale = jnp.minimum(1.0, (LAM * sqrtw) / (norm + 1e-12))
    ep = u * scale
    eproj_ref[...] = ep
    dual_ref[...] = sqrtw * ep


def _edge_stage(e, g, w2, W_eu, b_edge):
    return pl.pallas_call(
        _edge_body,
        grid=(E // TE,),
        in_specs=[pl.BlockSpec((TE, D), lambda i: (i, 0)),
                  pl.BlockSpec((TE, D), lambda i: (i, 0)),
                  pl.BlockSpec((TE, 1), lambda i: (i, 0)),
                  pl.BlockSpec((D, D), lambda i: (0, 0)),
                  pl.BlockSpec((1, D), lambda i: (0, 0))],
        out_specs=[pl.BlockSpec((TE, D), lambda i: (i, 0)),
                   pl.BlockSpec((TE, D), lambda i: (i, 0))],
        out_shape=[jax.ShapeDtypeStruct((E, D), jnp.float32),
                   jax.ShapeDtypeStruct((E, D), jnp.float32)],
    )(e, g, w2, W_eu, b_edge)


def _node_body(h_ref, x_ref, aggp_ref, wnf, wres, wagg, wf1, wf2,
               bn, bf1, bf2, wea, o_ref, hw_ref):
    agg = aggp_ref[0] + aggp_ref[1]
    ni = (_dot(h_ref[...], wnf[...]) + _dot(x_ref[...], wres[...])
          + _dot(agg, wagg[...]) + bn[...])
    hid = jax.nn.silu(_dot(ni, wf1[...]) + bf1[...])
    h_new = _dot(hid, wf2[...]) + bf2[...]
    o_ref[...] = h_new
    hw_ref[...] = _dot(h_new, wea[...])


def _node_stage(h, x, aggp, p, W_ea_next):
    wspec = pl.BlockSpec((D, D), lambda i: (0, 0))
    bspec = pl.BlockSpec((1, D), lambda i: (0, 0))
    bn = (p['b_nf'] + p['b_res'] + p['b_agg']).reshape(1, D)
    return pl.pallas_call(
        _node_body,
        grid=(N // TN,),
        in_specs=[pl.BlockSpec((TN, D), lambda i: (i, 0)),
                  pl.BlockSpec((TN, D), lambda i: (i, 0)),
                  pl.BlockSpec((NC, TN, D), lambda i: (0, i, 0)),
                  wspec, wspec, wspec, wspec, wspec,
                  bspec, bspec, bspec, wspec],
        out_specs=[pl.BlockSpec((TN, D), lambda i: (i, 0)),
                   pl.BlockSpec((TN, D), lambda i: (i, 0))],
        out_shape=[jax.ShapeDtypeStruct((N, D), jnp.float32),
                   jax.ShapeDtypeStruct((N, D), jnp.float32)],
    )(h, x, aggp, p['W_nf'], p['W_res'], p['W_agg'], p['W_f1'], p['W_f2'],
      bn, p['b_f1'].reshape(1, D), p['b_f2'].reshape(1, D), W_ea_next)


def kernel(h, e, edge_index, w, x, params):
    src = edge_index[0]
    dst = edge_index[1]
    src3 = src.reshape(NW, NCH, CH)
    dst3 = dst.reshape(NW, NCH, CH)
    dual_shape = (NW, NCH, CH, D)
    w2 = w.reshape(E, 1)
    zeros = jnp.zeros((N, D), jnp.float32)
    hW = _hw_mm(h, params[0]['W_ea'])
    nlayers = len(params)
    for li, p in enumerate(params):
        g = _gather(hW, src3, dst3)
        b_edge = (p['b_eu'] + p['b_ea']).reshape(1, D)
        e, dual = _edge_stage(e, g, w2, p['W_eu'], b_edge)
        aggp = _scatter(dual.reshape(dual_shape), dst3, zeros)
        if li + 1 < nlayers:
            h, hW = _node_stage(h, x, aggp, p, params[li + 1]['W_ea'])
        else:
            h, _ = _node_stage(h, x, aggp, p, p['W_ea'])
    return (h, e)


# speedup vs baseline: 4.3483x; 1.0455x over previous
import functools

import jax
import jax.numpy as jnp
from jax import lax
from jax.experimental import pallas as pl
from jax.experimental.pallas import tpu as pltpu
from jax.experimental.pallas import tpu_sc as plsc

N = 10000
E = 320000
D = 128
LAM = 1.0

NC = 2
NS = 16
NW = NC * NS
EH = E // 2
CH = 40
NCH = (EH // NW) // CH
NPAIR = (NCH - 1) // 2

_MESH = plsc.VectorSubcoreMesh(core_axis_name="c", subcore_axis_name="s")


def _gather_body(tbl, src3, dst3, g_hbm, idxs, idxd, bufs, bufd, bufg,
                 sem_s0, sem_s1, sem_d0, sem_d1, sem_w):
    cid = lax.axis_index("c")
    sid = lax.axis_index("s")
    wid = cid * NS + sid
    pltpu.sync_copy(src3.at[wid], idxs)
    pltpu.sync_copy(dst3.at[wid], idxd)
    sems = (sem_s0, sem_s1)
    semd = (sem_d0, sem_d1)

    def issue(c, k):
        pltpu.async_copy(tbl.at[idxs.at[c]], bufs.at[k], sems[k])
        pltpu.async_copy(tbl.at[idxd.at[c]], bufd.at[k], semd[k])

    def wait_gather(k):
        pltpu.make_async_copy(tbl.at[idxs.at[0]], bufs.at[k], sems[k]).wait()
        pltpu.make_async_copy(tbl.at[idxd.at[0]], bufd.at[k], semd[k]).wait()

    def sub(k):
        bs, bd, bg = bufs.at[k], bufd.at[k], bufg.at[k]

        def row(r, carry):
            for j in range(D // 16):
                sl = pl.ds(j * 16, 16)
                bg[r, sl] = bs[r, sl] - bd[r, sl]
            return carry

        lax.fori_loop(0, CH, row, 0)

    def write(c, k):
        off = (wid * NCH + c) * CH
        pltpu.async_copy(bufg.at[k], g_hbm.at[pl.ds(off, CH)], sem_w)

    def wait_write():
        pltpu.make_async_copy(bufg.at[0], g_hbm.at[pl.ds(0, CH)],
                              sem_w).wait()

    issue(0, 0)
    issue(1, 1)

    def body(i, carry):
        c0 = 2 * i
        wait_gather(0)

        @pl.when(i > 0)
        def _():
            wait_write()
            wait_write()

        sub(0)
        issue(c0 + 2, 0)
        write(c0, 0)
        wait_gather(1)
        sub(1)

        @pl.when(c0 + 3 < NCH)
        def _():
            issue(c0 + 3, 1)

        write(c0 + 1, 1)
        return carry

    lax.fori_loop(0, NPAIR, body, 0)
    wait_gather(0)
    wait_write()
    wait_write()
    sub(0)
    write(NCH - 1, 0)
    wait_write()


_gather = pl.kernel(
    _gather_body,
    out_type=jax.ShapeDtypeStruct((EH, D), jnp.float32),
    mesh=_MESH,
    scratch_types=[
        pltpu.VMEM((NCH, CH), jnp.int32),
        pltpu.VMEM((NCH, CH), jnp.int32),
        pltpu.VMEM((2, CH, D), jnp.float32),
        pltpu.VMEM((2, CH, D), jnp.float32),
        pltpu.VMEM((2, CH, D), jnp.float32),
        pltpu.SemaphoreType.DMA,
        pltpu.SemaphoreType.DMA,
        pltpu.SemaphoreType.DMA,
        pltpu.SemaphoreType.DMA,
        pltpu.SemaphoreType.DMA,
    ],
)


def _scatter_body(dual4, dst3, zeros_hbm, out_hbm, idxd, buf, acc,
                  sem_r0, sem_r1):
    cid = lax.axis_index("c")
    sid = lax.axis_index("s")
    wid = cid * NS + sid

    @pl.when(sid == 0)
    def _():
        pltpu.sync_copy(zeros_hbm, acc)

    pltpu.sync_copy(dst3.at[wid], idxd)
    plsc.subcore_barrier()
    semr = (sem_r0, sem_r1)

    def issue(c, k):
        pltpu.async_copy(dual4.at[wid, c], buf.at[k], semr[k])

    def wait_read(k):
        pltpu.make_async_copy(dual4.at[wid, 0], buf.at[k], semr[k]).wait()

    def scat(c, k):
        pltpu.sync_copy(buf.at[k], acc.at[idxd.at[c]], add=True)

    issue(0, 0)
    issue(1, 1)

    def body(i, carry):
        c0 = 2 * i
        wait_read(0)
        scat(c0, 0)
        issue(c0 + 2, 0)
        wait_read(1)
        scat(c0 + 1, 1)

        @pl.when(c0 + 3 < NCH)
        def _():
            issue(c0 + 3, 1)

        return carry

    lax.fori_loop(0, NPAIR, body, 0)
    wait_read(0)
    scat(NCH - 1, 0)
    plsc.subcore_barrier()

    @pl.when(sid < 10)
    def _():
        r0 = sid * 1000
        pltpu.sync_copy(acc.at[pl.ds(r0, 1000)],
                        out_hbm.at[cid].at[pl.ds(r0, 1000)])


_scatter = pl.kernel(
    _scatter_body,
    out_type=jax.ShapeDtypeStruct((NC, N, D), jnp.float32),
    mesh=_MESH,
    scratch_types=[
        pltpu.VMEM((NCH, CH), jnp.int32),
        pltpu.VMEM((2, CH, D), jnp.float32),
        pltpu.VMEM_SHARED((N, D), jnp.float32),
        pltpu.SemaphoreType.DMA,
        pltpu.SemaphoreType.DMA,
    ],
)


def _dot(a, b):
    return lax.dot_general(a, b, (((1,), (0,)), ((), ())),
                           preferred_element_type=jnp.float32)


def _hw_body(h_ref, w_ref, o_ref):
    o_ref[...] = _dot(h_ref[...], w_ref[...])


TN = 1000


def _hw_mm(h, W):
    return pl.pallas_call(
        _hw_body,
        grid=(N // TN,),
        in_specs=[pl.BlockSpec((TN, D), lambda i: (i, 0)),
                  pl.BlockSpec((D, D), lambda i: (0, 0))],
        out_specs=pl.BlockSpec((TN, D), lambda i: (i, 0)),
        out_shape=jax.ShapeDtypeStruct((N, D), jnp.float32),
    )(h, W)


TE = 2000


def _edge_body(first, last, e_ref, g_ref, w_ref, weu_ref, b_ref, *outs):
    sqrtw = jnp.sqrt(w_ref[...])
    if first:
        ein = e_ref[...]
    else:
        ein = e_ref[...] * (1.0 / (sqrtw + 1e-30))
    u = _dot(ein, weu_ref[...])
    u = u + b_ref[...] + sqrtw * g_ref[...]
    norm = jnp.sqrt(jnp.sum(u * u, axis=1, keepdims=True))
    scale = jnp.minimum(1.0, (LAM * sqrtw) / (norm + 1e-12))
    ep = u * scale
    if last:
        outs[0][...] = ep
        outs[1][...] = sqrtw * ep
    else:
        outs[0][...] = sqrtw * ep


def _edge_stage(first, last, half, e_state, g, w2, W_eu, b_edge):
    base = half * (EH // TE)
    espec = (pl.BlockSpec((TE, D), lambda i: (i + base, 0)) if first
             else pl.BlockSpec((TE, D), lambda i: (i, 0)))
    out_specs = [pl.BlockSpec((TE, D), lambda i: (i, 0))]
    out_shape = [jax.ShapeDtypeStruct((EH, D), jnp.float32)]
    if last:
        out_specs = out_specs * 2
        out_shape = out_shape * 2
    return pl.pallas_call(
        functools.partial(_edge_body, first, last),
        grid=(EH // TE,),
        in_specs=[espec,
                  pl.BlockSpec((TE, D), lambda i: (i, 0)),
                  pl.BlockSpec((TE, 1), lambda i: (i + base, 0)),
                  pl.BlockSpec((D, D), lambda i: (0, 0)),
                  pl.BlockSpec((1, D), lambda i: (0, 0))],
        out_specs=out_specs,
        out_shape=out_shape,
    )(e_state, g, w2, W_eu, b_edge)


def _node_body(h_ref, x_ref, aggA_ref, aggB_ref, wnf, wres, wagg, wf1, wf2,
               bn, bf1, bf2, wea, o_ref, hw_ref):
    agg = (aggA_ref[0] + aggA_ref[1]) + (aggB_ref[0] + aggB_ref[1])
    ni = (_dot(h_ref[...], wnf[...]) + _dot(x_ref[...], wres[...])
          + _dot(agg, wagg[...]) + bn[...])
    hid = jax.nn.silu(_dot(ni, wf1[...]) + bf1[...])
    h_new = _dot(hid, wf2[...]) + bf2[...]
    o_ref[...] = h_new
    hw_ref[...] = _dot(h_new, wea[...])


def _node_stage(h, x, aggA, aggB, p, W_ea_next):
    wspec = pl.BlockSpec((D, D), lambda i: (0, 0))
    bspec = pl.BlockSpec((1, D), lambda i: (0, 0))
    aspec = pl.BlockSpec((NC, TN, D), lambda i: (0, i, 0))
    bn = (p['b_nf'] + p['b_res'] + p['b_agg']).reshape(1, D)
    return pl.pallas_call(
        _node_body,
        grid=(N // TN,),
        in_specs=[pl.BlockSpec((TN, D), lambda i: (i, 0)),
                  pl.BlockSpec((TN, D), lambda i: (i, 0)),
                  aspec, aspec,
                  wspec, wspec, wspec, wspec, wspec,
                  bspec, bspec, bspec, wspec],
        out_specs=[pl.BlockSpec((TN, D), lambda i: (i, 0)),
                   pl.BlockSpec((TN, D), lambda i: (i, 0))],
        out_shape=[jax.ShapeDtypeStruct((N, D), jnp.float32),
                   jax.ShapeDtypeStruct((N, D), jnp.float32)],
    )(h, x, aggA, aggB, p['W_nf'], p['W_res'], p['W_agg'], p['W_f1'],
      p['W_f2'], bn, p['b_f1'].reshape(1, D), p['b_f2'].reshape(1, D),
      W_ea_next)


def kernel(h, e, edge_index, w, x, params):
    src = edge_index[0]
    dst = edge_index[1]
    idx3 = []
    for half in range(2):
        sl = slice(half * EH, (half + 1) * EH)
        idx3.append((src[sl].reshape(NW, NCH, CH),
                     dst[sl].reshape(NW, NCH, CH)))
    w2 = w.reshape(E, 1)
    zeros = jnp.zeros((N, D), jnp.float32)
    hW = _hw_mm(h, params[0]['W_ea'])
    nlayers = len(params)
    estate = [e, e]
    for li, p in enumerate(params):
        first = li == 0
        last = li + 1 == nlayers
        b_edge = (p['b_eu'] + p['b_ea']).reshape(1, D)
        g = [_gather(hW, s3, d3) for (s3, d3) in idx3]
        outs = [_edge_stage(first, last, half, estate[half], g[half], w2,
                            p['W_eu'], b_edge)
                for half in range(2)]
        if last:
            eproj = [o[0] for o in outs]
            dual = [o[1] for o in outs]
        else:
            dual = [o[0] for o in outs]
        aggp = [_scatter(dual[half].reshape(NW, NCH, CH, D), idx3[half][1],
                         zeros)
                for half in range(2)]
        W_ea_next = params[li + 1]['W_ea'] if not last else p['W_ea']
        h, hW = _node_stage(h, x, aggp[0], aggp[1], p, W_ea_next)
        estate = dual
    return (h, jnp.concatenate(eproj, axis=0))


# speedup vs baseline: 4.7979x; 1.1034x over previous
import functools

import jax
import jax.numpy as jnp
from jax import lax
from jax.experimental import pallas as pl
from jax.experimental.pallas import tpu as pltpu
from jax.experimental.pallas import tpu_sc as plsc

N = 10000
E = 320000
D = 128
LAM = 1.0

NC = 2
NS = 16
NW = NC * NS
EH = E // 2
CH = 40
NCH = (EH // NW) // CH
NQUAD = NCH // 4

_MESH = plsc.VectorSubcoreMesh(core_axis_name="c", subcore_axis_name="s")


def _gather_body(tbl, src3, dst3, g_hbm, idxs, idxd, bufs, bufd, bufg,
                 sem_s0, sem_s1, sem_s2, sem_s3,
                 sem_d0, sem_d1, sem_d2, sem_d3, sem_w):
    cid = lax.axis_index("c")
    sid = lax.axis_index("s")
    wid = cid * NS + sid
    pltpu.sync_copy(src3.at[wid], idxs)
    pltpu.sync_copy(dst3.at[wid], idxd)
    sems = (sem_s0, sem_s1, sem_s2, sem_s3)
    semd = (sem_d0, sem_d1, sem_d2, sem_d3)

    def issue(c, k):
        pltpu.async_copy(tbl.at[idxs.at[c]], bufs.at[k], sems[k])
        pltpu.async_copy(tbl.at[idxd.at[c]], bufd.at[k], semd[k])

    def wait_gather(k):
        pltpu.make_async_copy(tbl.at[idxs.at[0]], bufs.at[k], sems[k]).wait()
        pltpu.make_async_copy(tbl.at[idxd.at[0]], bufd.at[k], semd[k]).wait()

    def sub(k):
        bs, bd, bg = bufs.at[k], bufd.at[k], bufg.at[k]

        def row(r, carry):
            for j in range(D // 16):
                sl = pl.ds(j * 16, 16)
                bg[r, sl] = bs[r, sl] - bd[r, sl]
            return carry

        lax.fori_loop(0, CH, row, 0)

    def write(c, k):
        off = (wid * NCH + c) * CH
        pltpu.async_copy(bufg.at[k], g_hbm.at[pl.ds(off, CH)], sem_w)

    def wait_write():
        pltpu.make_async_copy(bufg.at[0], g_hbm.at[pl.ds(0, CH)],
                              sem_w).wait()

    for k in range(4):
        issue(k, k)

    def body(i, carry):
        c0 = 4 * i

        @pl.when(i > 0)
        def _():
            for _k in range(4):
                wait_write()

        for k in range(4):
            wait_gather(k)
            sub(k)

            if k == 0:
                issue(c0 + 4, k)
            else:
                @pl.when(c0 + 4 + k < NCH)
                def _(k=k):
                    issue(c0 + 4 + k, k)

            write(c0 + k, k)
        return carry

    lax.fori_loop(0, NQUAD, body, 0)
    wait_gather(0)
    for _k in range(4):
        wait_write()
    sub(0)
    write(NCH - 1, 0)
    wait_write()


_gather = pl.kernel(
    _gather_body,
    out_type=jax.ShapeDtypeStruct((EH, D), jnp.float32),
    mesh=_MESH,
    scratch_types=[
        pltpu.VMEM((NCH, CH), jnp.int32),
        pltpu.VMEM((NCH, CH), jnp.int32),
        pltpu.VMEM((4, CH, D), jnp.float32),
        pltpu.VMEM((4, CH, D), jnp.float32),
        pltpu.VMEM((4, CH, D), jnp.float32),
        pltpu.SemaphoreType.DMA,
        pltpu.SemaphoreType.DMA,
        pltpu.SemaphoreType.DMA,
        pltpu.SemaphoreType.DMA,
        pltpu.SemaphoreType.DMA,
        pltpu.SemaphoreType.DMA,
        pltpu.SemaphoreType.DMA,
        pltpu.SemaphoreType.DMA,
        pltpu.SemaphoreType.DMA,
    ],
)


def _scatter_body(dual4, dst3, zeros_hbm, out_hbm, idxd, buf, acc,
                  sem_r0, sem_r1, sem_r2, sem_r3):
    cid = lax.axis_index("c")
    sid = lax.axis_index("s")
    wid = cid * NS + sid

    @pl.when(sid == 0)
    def _():
        pltpu.sync_copy(zeros_hbm, acc)

    pltpu.sync_copy(dst3.at[wid], idxd)
    plsc.subcore_barrier()
    semr = (sem_r0, sem_r1, sem_r2, sem_r3)

    def issue(c, k):
        pltpu.async_copy(dual4.at[wid, c], buf.at[k], semr[k])

    def wait_read(k):
        pltpu.make_async_copy(dual4.at[wid, 0], buf.at[k], semr[k]).wait()

    def scat(c, k):
        pltpu.sync_copy(buf.at[k], acc.at[idxd.at[c]], add=True)

    for k in range(4):
        issue(k, k)

    def body(i, carry):
        c0 = 4 * i
        for k in range(4):
            wait_read(k)
            scat(c0 + k, k)

            if k == 0:
                issue(c0 + 4, k)
            else:
                @pl.when(c0 + 4 + k < NCH)
                def _(k=k):
                    issue(c0 + 4 + k, k)

        return carry

    lax.fori_loop(0, NQUAD, body, 0)
    wait_read(0)
    scat(NCH - 1, 0)
    plsc.subcore_barrier()

    @pl.when(sid < 10)
    def _():
        r0 = sid * 1000
        pltpu.sync_copy(acc.at[pl.ds(r0, 1000)],
                        out_hbm.at[cid].at[pl.ds(r0, 1000)])


_scatter = pl.kernel(
    _scatter_body,
    out_type=jax.ShapeDtypeStruct((NC, N, D), jnp.float32),
    mesh=_MESH,
    scratch_types=[
        pltpu.VMEM((NCH, CH), jnp.int32),
        pltpu.VMEM((4, CH, D), jnp.float32),
        pltpu.VMEM_SHARED((N, D), jnp.float32),
        pltpu.SemaphoreType.DMA,
        pltpu.SemaphoreType.DMA,
        pltpu.SemaphoreType.DMA,
        pltpu.SemaphoreType.DMA,
    ],
)


def _dot(a, b):
    return lax.dot_general(a, b, (((1,), (0,)), ((), ())),
                           preferred_element_type=jnp.float32)


def _hw_body(h_ref, w_ref, o_ref):
    o_ref[...] = _dot(h_ref[...], w_ref[...])


TN = 1000


def _hw_mm(h, W):
    return pl.pallas_call(
        _hw_body,
        grid=(N // TN,),
        in_specs=[pl.BlockSpec((TN, D), lambda i: (i, 0)),
                  pl.BlockSpec((D, D), lambda i: (0, 0))],
        out_specs=pl.BlockSpec((TN, D), lambda i: (i, 0)),
        out_shape=jax.ShapeDtypeStruct((N, D), jnp.float32),
    )(h, W)


TE = 4000


def _edge_body(first, last, e_ref, g_ref, w_ref, weu_ref, b_ref, *outs):
    sqrtw = jnp.sqrt(w_ref[...])
    if first:
        ein = e_ref[...]
    else:
        ein = e_ref[...] * (1.0 / (sqrtw + 1e-30))
    u = _dot(ein, weu_ref[...])
    u = u + b_ref[...] + sqrtw * g_ref[...]
    norm = jnp.sqrt(jnp.sum(u * u, axis=1, keepdims=True))
    scale = jnp.minimum(1.0, (LAM * sqrtw) / (norm + 1e-12))
    ep = u * scale
    if last:
        outs[0][...] = ep
        outs[1][...] = sqrtw * ep
    else:
        outs[0][...] = sqrtw * ep


def _edge_stage(first, last, half, e_state, g, w2, W_eu, b_edge):
    base = half * (EH // TE)
    espec = (pl.BlockSpec((TE, D), lambda i: (i + base, 0)) if first
             else pl.BlockSpec((TE, D), lambda i: (i, 0)))
    out_specs = [pl.BlockSpec((TE, D), lambda i: (i, 0))]
    out_shape = [jax.ShapeDtypeStruct((EH, D), jnp.float32)]
    if last:
        out_specs = out_specs * 2
        out_shape = out_shape * 2
    return pl.pallas_call(
        functools.partial(_edge_body, first, last),
        grid=(EH // TE,),
        in_specs=[espec,
                  pl.BlockSpec((TE, D), lambda i: (i, 0)),
                  pl.BlockSpec((TE, 1), lambda i: (i + base, 0)),
                  pl.BlockSpec((D, D), lambda i: (0, 0)),
                  pl.BlockSpec((1, D), lambda i: (0, 0))],
        out_specs=out_specs,
        out_shape=out_shape,
    )(e_state, g, w2, W_eu, b_edge)


def _node_body(h_ref, x_ref, aggA_ref, aggB_ref, wnf, wres, wagg, wf1, wf2,
               bn, bf1, bf2, wea, o_ref, hw_ref):
    agg = (aggA_ref[0] + aggA_ref[1]) + (aggB_ref[0] + aggB_ref[1])
    ni = (_dot(h_ref[...], wnf[...]) + _dot(x_ref[...], wres[...])
          + _dot(agg, wagg[...]) + bn[...])
    hid = jax.nn.silu(_dot(ni, wf1[...]) + bf1[...])
    h_new = _dot(hid, wf2[...]) + bf2[...]
    o_ref[...] = h_new
    hw_ref[...] = _dot(h_new, wea[...])


def _node_stage(h, x, aggA, aggB, p, W_ea_next):
    wspec = pl.BlockSpec((D, D), lambda i: (0, 0))
    bspec = pl.BlockSpec((1, D), lambda i: (0, 0))
    aspec = pl.BlockSpec((NC, TN, D), lambda i: (0, i, 0))
    bn = (p['b_nf'] + p['b_res'] + p['b_agg']).reshape(1, D)
    return pl.pallas_call(
        _node_body,
        grid=(N // TN,),
        in_specs=[pl.BlockSpec((TN, D), lambda i: (i, 0)),
                  pl.BlockSpec((TN, D), lambda i: (i, 0)),
                  aspec, aspec,
                  wspec, wspec, wspec, wspec, wspec,
                  bspec, bspec, bspec, wspec],
        out_specs=[pl.BlockSpec((TN, D), lambda i: (i, 0)),
                   pl.BlockSpec((TN, D), lambda i: (i, 0))],
        out_shape=[jax.ShapeDtypeStruct((N, D), jnp.float32),
                   jax.ShapeDtypeStruct((N, D), jnp.float32)],
    )(h, x, aggA, aggB, p['W_nf'], p['W_res'], p['W_agg'], p['W_f1'],
      p['W_f2'], bn, p['b_f1'].reshape(1, D), p['b_f2'].reshape(1, D),
      W_ea_next)


def kernel(h, e, edge_index, w, x, params):
    src = edge_index[0]
    dst = edge_index[1]
    idx3 = []
    for half in range(2):
        sl = slice(half * EH, (half + 1) * EH)
        idx3.append((src[sl].reshape(NW, NCH, CH),
                     dst[sl].reshape(NW, NCH, CH)))
    w2 = w.reshape(E, 1)
    zeros = jnp.zeros((N, D), jnp.float32)
    hW = _hw_mm(h, params[0]['W_ea'])
    nlayers = len(params)
    estate = [e, e]
    for li, p in enumerate(params):
        first = li == 0
        last = li + 1 == nlayers
        b_edge = (p['b_eu'] + p['b_ea']).reshape(1, D)
        g = [_gather(hW, s3, d3) for (s3, d3) in idx3]
        outs = [_edge_stage(first, last, half, estate[half], g[half], w2,
                            p['W_eu'], b_edge)
                for half in range(2)]
        if last:
            eproj = [o[0] for o in outs]
            dual = [o[1] for o in outs]
        else:
            dual = [o[0] for o in outs]
        aggp = [_scatter(dual[half].reshape(NW, NCH, CH, D), idx3[half][1],
                         zeros)
                for half in range(2)]
        W_ea_next = params[li + 1]['W_ea'] if not last else p['W_ea']
        h, hW = _node_stage(h, x, aggp[0], aggp[1], p, W_ea_next)
        estate = dual
    return (h, jnp.concatenate(eproj, axis=0))


# speedup vs baseline: 4.8067x; 1.0019x over previous
import functools

import jax
import jax.numpy as jnp
from jax import lax
from jax.experimental import pallas as pl
from jax.experimental.pallas import tpu as pltpu
from jax.experimental.pallas import tpu_sc as plsc

N = 10000
E = 320000
D = 128
LAM = 1.0

NC = 2
NS = 16
NW = NC * NS
EH = E // 2
CH = 40
NCH = (EH // NW) // CH
NQUAD = NCH // 4

_MESH = plsc.VectorSubcoreMesh(core_axis_name="c", subcore_axis_name="s")


def _gather_body(tbl, src3, dst3, g_hbm, idxs, idxd, bufs, bufd, bufg,
                 sem_s0, sem_s1, sem_s2, sem_s3,
                 sem_d0, sem_d1, sem_d2, sem_d3, sem_w):
    cid = lax.axis_index("c")
    sid = lax.axis_index("s")
    wid = cid * NS + sid
    pltpu.sync_copy(src3.at[wid], idxs)
    pltpu.sync_copy(dst3.at[wid], idxd)
    sems = (sem_s0, sem_s1, sem_s2, sem_s3)
    semd = (sem_d0, sem_d1, sem_d2, sem_d3)

    def issue(c, k):
        pltpu.async_copy(tbl.at[idxs.at[c]], bufs.at[k], sems[k])
        pltpu.async_copy(tbl.at[idxd.at[c]], bufd.at[k], semd[k])

    def wait_gather(k):
        pltpu.make_async_copy(tbl.at[idxs.at[0]], bufs.at[k], sems[k]).wait()
        pltpu.make_async_copy(tbl.at[idxd.at[0]], bufd.at[k], semd[k]).wait()

    def sub(k):
        bs, bd, bg = bufs.at[k], bufd.at[k], bufg.at[k]

        def row(r, carry):
            for j in range(D // 16):
                sl = pl.ds(j * 16, 16)
                bg[r, sl] = bs[r, sl] - bd[r, sl]
            return carry

        lax.fori_loop(0, CH, row, 0)

    def write(c, k):
        off = (wid * NCH + c) * CH
        pltpu.async_copy(bufg.at[k], g_hbm.at[pl.ds(off, CH)], sem_w)

    def wait_write():
        pltpu.make_async_copy(bufg.at[0], g_hbm.at[pl.ds(0, CH)],
                              sem_w).wait()

    for k in range(4):
        issue(k, k)

    def body(i, carry):
        c0 = 4 * i

        @pl.when(i > 0)
        def _():
            for _k in range(4):
                wait_write()

        for k in range(4):
            wait_gather(k)
            sub(k)

            if k == 0:
                issue(c0 + 4, k)
            else:
                @pl.when(c0 + 4 + k < NCH)
                def _(k=k):
                    issue(c0 + 4 + k, k)

            write(c0 + k, k)
        return carry

    lax.fori_loop(0, NQUAD, body, 0)
    wait_gather(0)
    for _k in range(4):
        wait_write()
    sub(0)
    write(NCH - 1, 0)
    wait_write()


_gather = pl.kernel(
    _gather_body,
    out_type=jax.ShapeDtypeStruct((EH, D), jnp.float32),
    mesh=_MESH,
    scratch_types=[
        pltpu.VMEM((NCH, CH), jnp.int32),
        pltpu.VMEM((NCH, CH), jnp.int32),
        pltpu.VMEM((4, CH, D), jnp.float32),
        pltpu.VMEM((4, CH, D), jnp.float32),
        pltpu.VMEM((4, CH, D), jnp.float32),
        pltpu.SemaphoreType.DMA,
        pltpu.SemaphoreType.DMA,
        pltpu.SemaphoreType.DMA,
        pltpu.SemaphoreType.DMA,
        pltpu.SemaphoreType.DMA,
        pltpu.SemaphoreType.DMA,
        pltpu.SemaphoreType.DMA,
        pltpu.SemaphoreType.DMA,
        pltpu.SemaphoreType.DMA,
    ],
)


def _scatter_body(dual4, dst3, zeros_hbm, out_hbm, idxd, buf, acc,
                  sem_r0, sem_r1, sem_r2, sem_r3):
    cid = lax.axis_index("c")
    sid = lax.axis_index("s")
    wid = cid * NS + sid

    @pl.when(sid == 0)
    def _():
        pltpu.sync_copy(zeros_hbm, acc)

    pltpu.sync_copy(dst3.at[wid], idxd)
    plsc.subcore_barrier()
    semr = (sem_r0, sem_r1, sem_r2, sem_r3)

    def issue(c, k):
        pltpu.async_copy(dual4.at[wid, c], buf.at[k], semr[k])

    def wait_read(k):
        pltpu.make_async_copy(dual4.at[wid, 0], buf.at[k], semr[k]).wait()

    def scat(c, k):
        pltpu.sync_copy(buf.at[k], acc.at[idxd.at[c]], add=True)

    for k in range(4):
        issue(k, k)

    def body(i, carry):
        c0 = 4 * i
        for k in range(4):
            wait_read(k)
            scat(c0 + k, k)

            if k == 0:
                issue(c0 + 4, k)
            else:
                @pl.when(c0 + 4 + k < NCH)
                def _(k=k):
                    issue(c0 + 4 + k, k)

        return carry

    lax.fori_loop(0, NQUAD, body, 0)
    wait_read(0)
    scat(NCH - 1, 0)
    plsc.subcore_barrier()

    @pl.when(sid < 10)
    def _():
        r0 = sid * 1000
        pltpu.sync_copy(acc.at[pl.ds(r0, 1000)],
                        out_hbm.at[cid].at[pl.ds(r0, 1000)])


_scatter = pl.kernel(
    _scatter_body,
    out_type=jax.ShapeDtypeStruct((NC, N, D), jnp.float32),
    mesh=_MESH,
    scratch_types=[
        pltpu.VMEM((NCH, CH), jnp.int32),
        pltpu.VMEM((4, CH, D), jnp.float32),
        pltpu.VMEM_SHARED((N, D), jnp.float32),
        pltpu.SemaphoreType.DMA,
        pltpu.SemaphoreType.DMA,
        pltpu.SemaphoreType.DMA,
        pltpu.SemaphoreType.DMA,
    ],
)


def _dot(a, b):
    return lax.dot_general(a, b, (((1,), (0,)), ((), ())),
                           preferred_element_type=jnp.float32)


def _hw_body(h_ref, w_ref, o_ref):
    o_ref[...] = _dot(h_ref[...], w_ref[...])


TN = 1000


def _hw_mm(h, W):
    return pl.pallas_call(
        _hw_body,
        grid=(N // TN,),
        in_specs=[pl.BlockSpec((TN, D), lambda i: (i, 0)),
                  pl.BlockSpec((D, D), lambda i: (0, 0))],
        out_specs=pl.BlockSpec((TN, D), lambda i: (i, 0)),
        out_shape=jax.ShapeDtypeStruct((N, D), jnp.float32),
    )(h, W)


TE = 4000


def _edge_body(first, last, e_ref, g_ref, w_ref, weu_ref, b_ref, *outs):
    sqrtw = jnp.sqrt(w_ref[...])
    if first:
        ein = e_ref[...]
    else:
        ein = e_ref[...] * (1.0 / (sqrtw + 1e-30))
    u = _dot(ein.astype(jnp.bfloat16), weu_ref[...].astype(jnp.bfloat16))
    u = u + b_ref[...] + sqrtw * g_ref[...]
    norm = jnp.sqrt(jnp.sum(u * u, axis=1, keepdims=True))
    scale = jnp.minimum(1.0, (LAM * sqrtw) / (norm + 1e-12))
    ep = u * scale
    if last:
        outs[0][...] = ep
        outs[1][...] = sqrtw * ep
    else:
        outs[0][...] = sqrtw * ep


def _edge_stage(first, last, half, e_state, g, w2, W_eu, b_edge):
    base = half * (EH // TE)
    espec = (pl.BlockSpec((TE, D), lambda i: (i + base, 0)) if first
             else pl.BlockSpec((TE, D), lambda i: (i, 0)))
    out_specs = [pl.BlockSpec((TE, D), lambda i: (i, 0))]
    out_shape = [jax.ShapeDtypeStruct((EH, D), jnp.float32)]
    if last:
        out_specs = out_specs * 2
        out_shape = out_shape * 2
    return pl.pallas_call(
        functools.partial(_edge_body, first, last),
        grid=(EH // TE,),
        in_specs=[espec,
                  pl.BlockSpec((TE, D), lambda i: (i, 0)),
                  pl.BlockSpec((TE, 1), lambda i: (i + base, 0)),
                  pl.BlockSpec((D, D), lambda i: (0, 0)),
                  pl.BlockSpec((1, D), lambda i: (0, 0))],
        out_specs=out_specs,
        out_shape=out_shape,
    )(e_state, g, w2, W_eu, b_edge)


def _node_body(h_ref, x_ref, aggA_ref, aggB_ref, wnf, wres, wagg, wf1, wf2,
               bn, bf1, bf2, wea, o_ref, hw_ref):
    agg = (aggA_ref[0] + aggA_ref[1]) + (aggB_ref[0] + aggB_ref[1])
    ni = (_dot(h_ref[...], wnf[...]) + _dot(x_ref[...], wres[...])
          + _dot(agg, wagg[...]) + bn[...])
    hid = jax.nn.silu(_dot(ni, wf1[...]) + bf1[...])
    h_new = _dot(hid, wf2[...]) + bf2[...]
    o_ref[...] = h_new
    hw_ref[...] = _dot(h_new, wea[...])


def _node_stage(h, x, aggA, aggB, p, W_ea_next):
    wspec = pl.BlockSpec((D, D), lambda i: (0, 0))
    bspec = pl.BlockSpec((1, D), lambda i: (0, 0))
    aspec = pl.BlockSpec((NC, TN, D), lambda i: (0, i, 0))
    bn = (p['b_nf'] + p['b_res'] + p['b_agg']).reshape(1, D)
    return pl.pallas_call(
        _node_body,
        grid=(N // TN,),
        in_specs=[pl.BlockSpec((TN, D), lambda i: (i, 0)),
                  pl.BlockSpec((TN, D), lambda i: (i, 0)),
                  aspec, aspec,
                  wspec, wspec, wspec, wspec, wspec,
                  bspec, bspec, bspec, wspec],
        out_specs=[pl.BlockSpec((TN, D), lambda i: (i, 0)),
                   pl.BlockSpec((TN, D), lambda i: (i, 0))],
        out_shape=[jax.ShapeDtypeStruct((N, D), jnp.float32),
                   jax.ShapeDtypeStruct((N, D), jnp.float32)],
    )(h, x, aggA, aggB, p['W_nf'], p['W_res'], p['W_agg'], p['W_f1'],
      p['W_f2'], bn, p['b_f1'].reshape(1, D), p['b_f2'].reshape(1, D),
      W_ea_next)


def kernel(h, e, edge_index, w, x, params):
    src = edge_index[0]
    dst = edge_index[1]
    idx3 = []
    for half in range(2):
        sl = slice(half * EH, (half + 1) * EH)
        idx3.append((src[sl].reshape(NW, NCH, CH),
                     dst[sl].reshape(NW, NCH, CH)))
    w2 = w.reshape(E, 1)
    zeros = jnp.zeros((N, D), jnp.float32)
    hW = _hw_mm(h, params[0]['W_ea'])
    nlayers = len(params)
    estate = [e, e]
    for li, p in enumerate(params):
        first = li == 0
        last = li + 1 == nlayers
        b_edge = (p['b_eu'] + p['b_ea']).reshape(1, D)
        g = [_gather(hW, s3, d3) for (s3, d3) in idx3]
        outs = [_edge_stage(first, last, half, estate[half], g[half], w2,
                            p['W_eu'], b_edge)
                for half in range(2)]
        if last:
            eproj = [o[0] for o in outs]
            dual = [o[1] for o in outs]
        else:
            dual = [o[0] for o in outs]
        aggp = [_scatter(dual[half].reshape(NW, NCH, CH, D), idx3[half][1],
                         zeros)
                for half in range(2)]
        W_ea_next = params[li + 1]['W_ea'] if not last else p['W_ea']
        h, hW = _node_stage(h, x, aggp[0], aggp[1], p, W_ea_next)
        estate = dual
    return (h, jnp.concatenate(eproj, axis=0))


# speedup vs baseline: 4.8069x; 1.0000x over previous
import functools

import jax
import jax.numpy as jnp
from jax import lax
from jax.experimental import pallas as pl
from jax.experimental.pallas import tpu as pltpu
from jax.experimental.pallas import tpu_sc as plsc

N = 10000
E = 320000
D = 128
LAM = 1.0

NC = 2
NS = 16
NW = NC * NS
EH = E // 2
CH = 40
NCH = (EH // NW) // CH
NQUAD = NCH // 4

_MESH = plsc.VectorSubcoreMesh(core_axis_name="c", subcore_axis_name="s")


def _gather_body(tbl, src3, dst3, g_hbm, idxs, idxd, bufs, bufd, bufg,
                 sem_s0, sem_s1, sem_s2, sem_s3,
                 sem_d0, sem_d1, sem_d2, sem_d3, sem_w):
    cid = lax.axis_index("c")
    sid = lax.axis_index("s")
    wid = cid * NS + sid
    pltpu.sync_copy(src3.at[wid], idxs)
    pltpu.sync_copy(dst3.at[wid], idxd)
    sems = (sem_s0, sem_s1, sem_s2, sem_s3)
    semd = (sem_d0, sem_d1, sem_d2, sem_d3)

    def issue(c, k):
        pltpu.async_copy(tbl.at[idxs.at[c]], bufs.at[k], sems[k])
        pltpu.async_copy(tbl.at[idxd.at[c]], bufd.at[k], semd[k])

    def wait_gather(k):
        pltpu.make_async_copy(tbl.at[idxs.at[0]], bufs.at[k], sems[k]).wait()
        pltpu.make_async_copy(tbl.at[idxd.at[0]], bufd.at[k], semd[k]).wait()

    def sub(k):
        bs, bd, bg = bufs.at[k], bufd.at[k], bufg.at[k]

        def row(r, carry):
            for j in range(D // 16):
                sl = pl.ds(j * 16, 16)
                bg[r, sl] = bs[r, sl] - bd[r, sl]
            return carry

        lax.fori_loop(0, CH, row, 0)

    def write(c, k):
        off = (wid * NCH + c) * CH
        pltpu.async_copy(bufg.at[k], g_hbm.at[pl.ds(off, CH)], sem_w)

    def wait_write():
        pltpu.make_async_copy(bufg.at[0], g_hbm.at[pl.ds(0, CH)],
                              sem_w).wait()

    for k in range(4):
        issue(k, k)

    def body(i, carry):
        c0 = 4 * i

        @pl.when(i > 0)
        def _():
            for _k in range(4):
                wait_write()

        for k in range(4):
            wait_gather(k)
            sub(k)

            if k == 0:
                issue(c0 + 4, k)
            else:
                @pl.when(c0 + 4 + k < NCH)
                def _(k=k):
                    issue(c0 + 4 + k, k)

            write(c0 + k, k)
        return carry

    lax.fori_loop(0, NQUAD, body, 0)
    wait_gather(0)
    for _k in range(4):
        wait_write()
    sub(0)
    write(NCH - 1, 0)
    wait_write()


_gather = pl.kernel(
    _gather_body,
    out_type=jax.ShapeDtypeStruct((EH, D), jnp.float32),
    mesh=_MESH,
    scratch_types=[
        pltpu.VMEM((NCH, CH), jnp.int32),
        pltpu.VMEM((NCH, CH), jnp.int32),
        pltpu.VMEM((4, CH, D), jnp.float32),
        pltpu.VMEM((4, CH, D), jnp.float32),
        pltpu.VMEM((4, CH, D), jnp.float32),
        pltpu.SemaphoreType.DMA,
        pltpu.SemaphoreType.DMA,
        pltpu.SemaphoreType.DMA,
        pltpu.SemaphoreType.DMA,
        pltpu.SemaphoreType.DMA,
        pltpu.SemaphoreType.DMA,
        pltpu.SemaphoreType.DMA,
        pltpu.SemaphoreType.DMA,
        pltpu.SemaphoreType.DMA,
    ],
)


def _scatter_body(dual4, dst3, zeros_hbm, out_hbm, idxd, buf, acc,
                  sem_r0, sem_r1, sem_r2, sem_r3):
    cid = lax.axis_index("c")
    sid = lax.axis_index("s")
    wid = cid * NS + sid

    @pl.when(sid == 0)
    def _():
        pltpu.sync_copy(zeros_hbm, acc)

    pltpu.sync_copy(dst3.at[wid], idxd)
    plsc.subcore_barrier()
    semr = (sem_r0, sem_r1, sem_r2, sem_r3)

    def issue(c, k):
        pltpu.async_copy(dual4.at[wid, c], buf.at[k], semr[k])

    def wait_read(k):
        pltpu.make_async_copy(dual4.at[wid, 0], buf.at[k], semr[k]).wait()

    def scat(c, k):
        pltpu.sync_copy(buf.at[k], acc.at[idxd.at[c]], add=True)

    for k in range(4):
        issue(k, k)

    def body(i, carry):
        c0 = 4 * i
        for k in range(4):
            wait_read(k)
            scat(c0 + k, k)

            if k == 0:
                issue(c0 + 4, k)
            else:
                @pl.when(c0 + 4 + k < NCH)
                def _(k=k):
                    issue(c0 + 4 + k, k)

        return carry

    lax.fori_loop(0, NQUAD, body, 0)
    wait_read(0)
    scat(NCH - 1, 0)
    plsc.subcore_barrier()

    @pl.when(sid < 10)
    def _():
        r0 = sid * 1000
        pltpu.sync_copy(acc.at[pl.ds(r0, 1000)],
                        out_hbm.at[cid].at[pl.ds(r0, 1000)])


_scatter = pl.kernel(
    _scatter_body,
    out_type=jax.ShapeDtypeStruct((NC, N, D), jnp.float32),
    mesh=_MESH,
    scratch_types=[
        pltpu.VMEM((NCH, CH), jnp.int32),
        pltpu.VMEM((4, CH, D), jnp.float32),
        pltpu.VMEM_SHARED((N, D), jnp.float32),
        pltpu.SemaphoreType.DMA,
        pltpu.SemaphoreType.DMA,
        pltpu.SemaphoreType.DMA,
        pltpu.SemaphoreType.DMA,
    ],
)


def _dot(a, b):
    return lax.dot_general(a, b, (((1,), (0,)), ((), ())),
                           preferred_element_type=jnp.float32)


def _hw_body(h_ref, w_ref, o_ref):
    o_ref[...] = _dot(h_ref[...], w_ref[...])


TN = 1000


def _hw_mm(h, W):
    return pl.pallas_call(
        _hw_body,
        grid=(N // TN,),
        in_specs=[pl.BlockSpec((TN, D), lambda i: (i, 0)),
                  pl.BlockSpec((D, D), lambda i: (0, 0))],
        out_specs=pl.BlockSpec((TN, D), lambda i: (i, 0)),
        out_shape=jax.ShapeDtypeStruct((N, D), jnp.float32),
    )(h, W)


TE = 4000


def _edge_body(first, last, e_ref, g_ref, w_ref, weu_ref, b_ref, *outs):
    sqrtw = jnp.sqrt(w_ref[...])
    if first:
        ein = e_ref[...]
    else:
        ein = e_ref[...] * (1.0 / (sqrtw + 1e-30))
    u = _dot(ein, weu_ref[...])
    u = u + b_ref[...] + sqrtw * g_ref[...]
    norm = jnp.sqrt(jnp.sum(u * u, axis=1, keepdims=True))
    scale = jnp.minimum(1.0, (LAM * sqrtw) / (norm + 1e-12))
    ep = u * scale
    if last:
        outs[0][...] = ep
        outs[1][...] = sqrtw * ep
    else:
        outs[0][...] = sqrtw * ep


def _edge_stage(first, last, half, e_state, g, w2, W_eu, b_edge):
    base = half * (EH // TE)
    espec = (pl.BlockSpec((TE, D), lambda i: (i + base, 0)) if first
             else pl.BlockSpec((TE, D), lambda i: (i, 0)))
    out_specs = [pl.BlockSpec((TE, D), lambda i: (i, 0))]
    out_shape = [jax.ShapeDtypeStruct((EH, D), jnp.float32)]
    if last:
        out_specs = out_specs * 2
        out_shape = out_shape * 2
    return pl.pallas_call(
        functools.partial(_edge_body, first, last),
        grid=(EH // TE,),
        in_specs=[espec,
                  pl.BlockSpec((TE, D), lambda i: (i, 0)),
                  pl.BlockSpec((TE, 1), lambda i: (i + base, 0)),
                  pl.BlockSpec((D, D), lambda i: (0, 0)),
                  pl.BlockSpec((1, D), lambda i: (0, 0))],
        out_specs=out_specs,
        out_shape=out_shape,
    )(e_state, g, w2, W_eu, b_edge)


def _node_body(h_ref, x_ref, aggA_ref, aggB_ref, wnf, wres, wagg, wf1, wf2,
               bn, bf1, bf2, wea, o_ref, hw_ref):
    agg = (aggA_ref[0] + aggA_ref[1]) + (aggB_ref[0] + aggB_ref[1])
    ni = (_dot(h_ref[...], wnf[...]) + _dot(x_ref[...], wres[...])
          + _dot(agg, wagg[...]) + bn[...])
    hid = jax.nn.silu(_dot(ni, wf1[...]) + bf1[...])
    h_new = _dot(hid, wf2[...]) + bf2[...]
    o_ref[...] = h_new
    hw_ref[...] = _dot(h_new, wea[...])


def _node_stage(h, x, aggA, aggB, p, W_ea_next):
    wspec = pl.BlockSpec((D, D), lambda i: (0, 0))
    bspec = pl.BlockSpec((1, D), lambda i: (0, 0))
    aspec = pl.BlockSpec((NC, TN, D), lambda i: (0, i, 0))
    bn = (p['b_nf'] + p['b_res'] + p['b_agg']).reshape(1, D)
    return pl.pallas_call(
        _node_body,
        grid=(N // TN,),
        in_specs=[pl.BlockSpec((TN, D), lambda i: (i, 0)),
                  pl.BlockSpec((TN, D), lambda i: (i, 0)),
                  aspec, aspec,
                  wspec, wspec, wspec, wspec, wspec,
                  bspec, bspec, bspec, wspec],
        out_specs=[pl.BlockSpec((TN, D), lambda i: (i, 0)),
                   pl.BlockSpec((TN, D), lambda i: (i, 0))],
        out_shape=[jax.ShapeDtypeStruct((N, D), jnp.float32),
                   jax.ShapeDtypeStruct((N, D), jnp.float32)],
    )(h, x, aggA, aggB, p['W_nf'], p['W_res'], p['W_agg'], p['W_f1'],
      p['W_f2'], bn, p['b_f1'].reshape(1, D), p['b_f2'].reshape(1, D),
      W_ea_next)


def kernel(h, e, edge_index, w, x, params):
    src = edge_index[0]
    dst = edge_index[1]
    idx3 = []
    for half in range(2):
        sl = slice(half * EH, (half + 1) * EH)
        idx3.append((src[sl].reshape(NW, NCH, CH),
                     dst[sl].reshape(NW, NCH, CH)))
    w2 = w.reshape(E, 1)
    zeros = jnp.zeros((N, D), jnp.float32)
    hW = _hw_mm(h, params[0]['W_ea'])
    nlayers = len(params)
    estate = [e, e]
    for li, p in enumerate(params):
        first = li == 0
        last = li + 1 == nlayers
        b_edge = (p['b_eu'] + p['b_ea']).reshape(1, D)
        g = [_gather(hW, s3, d3) for (s3, d3) in idx3]
        outs = [_edge_stage(first, last, half, estate[half], g[half], w2,
                            p['W_eu'], b_edge)
                for half in range(2)]
        if last:
            eproj = [o[0] for o in outs]
            dual = [o[1] for o in outs]
        else:
            dual = [o[0] for o in outs]
        aggp = [_scatter(dual[half].reshape(NW, NCH, CH, D), idx3[half][1],
                         zeros)
                for half in range(2)]
        W_ea_next = params[li + 1]['W_ea'] if not last else p['W_ea']
        h, hW = _node_stage(h, x, aggp[0], aggp[1], p, W_ea_next)
        estate = dual
    return (h, jnp.concatenate(eproj, axis=0))
